# baseline (speedup 1.0000x reference)
import functools

import jax
import jax.numpy as jnp
from jax import lax
from jax.experimental import pallas as pl
from jax.experimental.pallas import tpu as pltpu
from jax.experimental.pallas import tpu_sc as plsc

N = 10000
E = 320000
D = 128
S = 128
R = 3
M = 32
G = 64

NC = 2
NS = 16
NW = NC * NS

B_EDGE = 80
ECH = E // B_EDGE
WCH = 128
N_PAD = 10240
RPT = N_PAD // NS
ZROWS = B_EDGE
GCH = 16
NGRP = WCH // GCH
NBUF = 4
NITER = WCH // NBUF

TBLK = 2000
NBLK = N // TBLK


def _sc_scatter_round(msg, src2d, dst2d):
    mesh = plsc.VectorSubcoreMesh(core_axis_name="c", subcore_axis_name="s")

    @functools.partial(
        pl.kernel,
        out_type=jax.ShapeDtypeStruct((NC, N_PAD, S), jnp.float32),
        mesh=mesh,
        scratch_types=(
            [pltpu.VMEM((2, GCH, B_EDGE), jnp.int32),
             pltpu.VMEM((2, GCH, B_EDGE), jnp.int32)]
            + [pltpu.VMEM((B_EDGE, S), jnp.float32) for _ in range(NBUF)]
            + [pltpu.VMEM_SHARED((N_PAD, S), jnp.float32)]
            + [pltpu.SemaphoreType.DMA for _ in range(2 * NBUF + 1)]
        ),
    )
    def k(msg_hbm, src_hbm, dst_hbm, out_hbm, srcg, dstg, *rest):
        rows = rest[:NBUF]
        agg_sh = rest[NBUF]
        gsems = rest[NBUF + 1:2 * NBUF + 1]
        ssems = rest[2 * NBUF + 1:3 * NBUF + 1]
        isem = rest[3 * NBUF + 1]
        c = lax.axis_index("c")
        s = lax.axis_index("s")
        wid = c * NS + s
        base = wid * WCH
        wcount = jnp.minimum(WCH, ECH - base)
        niter = wcount // NBUF
        ngrp_w = wcount // GCH

        def zrow(i, _):
            def zcol(j, _):
                rows[0][i, pl.ds(j * 16, 16)] = jnp.zeros((16,), jnp.float32)
                return 0
            return lax.fori_loop(0, S // 16, zcol, 0)
        lax.fori_loop(0, B_EDGE, zrow, 0)

        pltpu.sync_copy(src_hbm.at[pl.ds(base, GCH)], srcg.at[0])
        pltpu.sync_copy(dst_hbm.at[pl.ds(base, GCH)], dstg.at[0])
        for b in range(1, NBUF):
            pltpu.async_copy(msg_hbm.at[srcg.at[0, b]], rows[b], gsems[b])

        for kk in range(RPT // ZROWS):
            pltpu.async_copy(rows[0],
                             agg_sh.at[pl.ds(s * RPT + kk * ZROWS, ZROWS)],
                             ssems[0])
        for kk in range(RPT // ZROWS):
            pltpu.make_async_copy(
                rows[0], agg_sh.at[pl.ds(s * RPT + kk * ZROWS, ZROWS)],
                ssems[0]).wait()
        pltpu.async_copy(msg_hbm.at[srcg.at[0, 0]], rows[0], gsems[0])
        plsc.subcore_barrier()

        def step(t, _):
            j0 = NBUF * t
            g = j0 // GCH
            slot = g % 2
            for b in range(NBUF):
                kk = (j0 + b) % GCH
                pltpu.make_async_copy(msg_hbm.at[srcg.at[slot, kk]], rows[b],
                                      gsems[b]).wait()
                pltpu.async_copy(rows[b], agg_sh.at[dstg.at[slot, kk]],
                                 ssems[b], add=True)

            @pl.when(j0 + NBUF < wcount)
            def _():
                jn0 = j0 + NBUF
                gn = jn0 // GCH
                slotn = gn % 2

                @pl.when(gn > g)
                def _():
                    nb = base + gn * GCH
                    pltpu.make_async_copy(src_hbm.at[pl.ds(nb, GCH)],
                                          srcg.at[slotn], isem).wait()
                    pltpu.make_async_copy(dst_hbm.at[pl.ds(nb, GCH)],
                                          dstg.at[slotn], isem).wait()

                for b in range(NBUF):
                    kk = (j0 + b) % GCH
                    kn = (jn0 + b) % GCH
                    pltpu.make_async_copy(rows[b],
                                          agg_sh.at[dstg.at[slot, kk]],
                                          ssems[b]).wait()
                    pltpu.async_copy(msg_hbm.at[srcg.at[slotn, kn]], rows[b],
                                     gsems[b])

            @pl.when(jnp.logical_and(j0 % GCH == 0, g + 1 < ngrp_w))
            def _():
                nb = base + (g + 1) * GCH
                pltpu.async_copy(src_hbm.at[pl.ds(nb, GCH)],
                                 srcg.at[(g + 1) % 2], isem)
                pltpu.async_copy(dst_hbm.at[pl.ds(nb, GCH)],
                                 dstg.at[(g + 1) % 2], isem)
            return 0
        lax.fori_loop(0, niter, step, 0)

        lslot = ((wcount - NBUF) // GCH) % 2
        for b in range(NBUF):
            pltpu.make_async_copy(
                rows[b], agg_sh.at[dstg.at[lslot, (wcount - NBUF + b) % GCH]],
                ssems[b]).wait()
        plsc.subcore_barrier()

        nwb = RPT // ZROWS

        def sl(kk):
            return pl.ds(s * RPT + kk * ZROWS, ZROWS)

        for kk in range(nwb):
            pltpu.async_copy(agg_sh.at[sl(kk)], out_hbm.at[c].at[sl(kk)],
                             ssems[kk % 2])
        for kk in range(nwb):
            pltpu.make_async_copy(agg_sh.at[sl(kk)], out_hbm.at[c].at[sl(kk)],
                                  ssems[kk % 2]).wait()

    return k(msg, src2d, dst2d)


def _tc_encode(x, W_in, b_in, W_msg0, b_msg0):
    def body(x_ref, wi_ref, bi_ref, wm_ref, bm_ref, st_ref, msg_ref):
        st = jnp.maximum(
            jnp.dot(x_ref[...], wi_ref[...], preferred_element_type=jnp.float32)
            + bi_ref[...], 0.0)
        st_ref[...] = st
        msg_ref[...] = jnp.maximum(
            jnp.dot(st, wm_ref[...], preferred_element_type=jnp.float32)
            + bm_ref[...], 0.0)

    return pl.pallas_call(
        body,
        grid=(NBLK,),
        in_specs=[
            pl.BlockSpec((TBLK, D), lambda i: (i, 0)),
            pl.BlockSpec((D, S), lambda i: (0, 0)),
            pl.BlockSpec((1, S), lambda i: (0, 0)),
            pl.BlockSpec((S, S), lambda i: (0, 0)),
            pl.BlockSpec((1, S), lambda i: (0, 0)),
        ],
        out_specs=[
            pl.BlockSpec((TBLK, S), lambda i: (i, 0)),
            pl.BlockSpec((TBLK, S), lambda i: (i, 0)),
        ],
        out_shape=[
            jax.ShapeDtypeStruct((N, S), jnp.float32),
            jax.ShapeDtypeStruct((N, S), jnp.float32),
        ],
    )(x, W_in, b_in, W_msg0, b_msg0)


def _tc_update(agg, state, W_upd_r, b_upd_r, W_msg_n, b_msg_n):
    def body(agg_ref, st_ref, wu_ref, bu_ref, wm_ref, bm_ref,
             stn_ref, msg_ref):
        a = agg_ref[0] + agg_ref[1]
        h = st_ref[...] + jnp.maximum(
            jnp.dot(a, wu_ref[...], preferred_element_type=jnp.float32)
            + bu_ref[...], 0.0)
        stn_ref[...] = h
        msg_ref[...] = jnp.maximum(
            jnp.dot(h, wm_ref[...], preferred_element_type=jnp.float32)
            + bm_ref[...], 0.0)

    return pl.pallas_call(
        body,
        grid=(NBLK,),
        in_specs=[
            pl.BlockSpec((NC, TBLK, S), lambda i: (0, i, 0)),
            pl.BlockSpec((TBLK, S), lambda i: (i, 0)),
            pl.BlockSpec((S, S), lambda i: (0, 0)),
            pl.BlockSpec((1, S), lambda i: (0, 0)),
            pl.BlockSpec((S, S), lambda i: (0, 0)),
            pl.BlockSpec((1, S), lambda i: (0, 0)),
        ],
        out_specs=[
            pl.BlockSpec((TBLK, S), lambda i: (i, 0)),
            pl.BlockSpec((TBLK, S), lambda i: (i, 0)),
        ],
        out_shape=[
            jax.ShapeDtypeStruct((N, S), jnp.float32),
            jax.ShapeDtypeStruct((N, S), jnp.float32),
        ],
    )(agg, state, W_upd_r, b_upd_r, W_msg_n, b_msg_n)


def _tc_final(agg, state, W_upd_r, b_upd_r, batch3d, W_mean, b_mean, W_lv, b_lv):
    def body(agg_ref, st_ref, wu_ref, bu_ref, bat_ref, wm_ref, bm_ref,
             wl_ref, bl_ref, mean_ref, std_ref, acc_ref):
        i = pl.program_id(0)
        a = agg_ref[0] + agg_ref[1]
        h = st_ref[...] + jnp.maximum(
            jnp.dot(a, wu_ref[...], preferred_element_type=jnp.float32)
            + bu_ref[...], 0.0)
        bat = bat_ref[0]
        gids = lax.broadcasted_iota(jnp.int32, (G, TBLK), 0)
        P = (gids == bat).astype(jnp.float32)
        contrib = jnp.dot(P, h, preferred_element_type=jnp.float32)

        @pl.when(i == 0)
        def _():
            acc_ref[...] = contrib

        @pl.when(i > 0)
        def _():
            acc_ref[...] = acc_ref[...] + contrib

        @pl.when(i == NBLK - 1)
        def _():
            g = acc_ref[...]
            mean_ref[...] = (
                jnp.dot(g, wm_ref[...], preferred_element_type=jnp.float32)
                + bm_ref[...])
            lv = jnp.clip(
                jnp.dot(g, wl_ref[...], preferred_element_type=jnp.float32)
                + bl_ref[...], -20.0, 2.0)
            std_ref[...] = jnp.exp(0.5 * lv)

    return pl.pallas_call(
        body,
        grid=(NBLK,),
        in_specs=[
            pl.BlockSpec((NC, TBLK, S), lambda i: (0, i, 0)),
            pl.BlockSpec((TBLK, S), lambda i: (i, 0)),
            pl.BlockSpec((S, S), lambda i: (0, 0)),
            pl.BlockSpec((1, S), lambda i: (0, 0)),
            pl.BlockSpec((1, 1, TBLK), lambda i: (i, 0, 0)),
            pl.BlockSpec((S, M), lambda i: (0, 0)),
            pl.BlockSpec((1, M), lambda i: (0, 0)),
            pl.BlockSpec((S, M), lambda i: (0, 0)),
            pl.BlockSpec((1, M), lambda i: (0, 0)),
        ],
        out_specs=[
            pl.BlockSpec((G, M), lambda i: (0, 0)),
            pl.BlockSpec((G, M), lambda i: (0, 0)),
        ],
        out_shape=[
            jax.ShapeDtypeStruct((G, M), jnp.float32),
            jax.ShapeDtypeStruct((G, M), jnp.float32),
        ],
        scratch_shapes=[pltpu.VMEM((G, S), jnp.float32)],
        compiler_params=pltpu.CompilerParams(
            dimension_semantics=("arbitrary",)),
    )(agg, state, W_upd_r, b_upd_r, batch3d, W_mean, b_mean, W_lv, b_lv)


def kernel(x, edge_index, batch, W_in, b_in, W_msg, b_msg, W_upd, b_upd,
           W_mean, b_mean, W_lv, b_lv):
    src2d = edge_index[0].reshape(ECH, B_EDGE)
    dst2d = edge_index[1].reshape(ECH, B_EDGE)
    batch3d = batch.reshape(NBLK, 1, TBLK)

    b_in2 = b_in.reshape(1, S)
    b_msg2 = b_msg.reshape(R, 1, S)
    b_upd2 = b_upd.reshape(R, 1, S)

    state, msg = _tc_encode(x, W_in, b_in2, W_msg[0], b_msg2[0])
    for r in range(R - 1):
        agg = _sc_scatter_round(msg, src2d, dst2d)
        state, msg = _tc_update(agg, state, W_upd[r], b_upd2[r],
                                W_msg[r + 1], b_msg2[r + 1])
    agg = _sc_scatter_round(msg, src2d, dst2d)
    mean, std = _tc_final(agg, state, W_upd[R - 1], b_upd2[R - 1], batch3d,
                          W_mean, b_mean.reshape(1, M), W_lv,
                          b_lv.reshape(1, M))
    return (mean, std)


# speedup vs baseline: 1.0280x; 1.0280x over previous
import functools

import jax
import jax.numpy as jnp
from jax import lax
from jax.experimental import pallas as pl
from jax.experimental.pallas import tpu as pltpu
from jax.experimental.pallas import tpu_sc as plsc

N = 10000
E = 320000
D = 128
S = 128
R = 3
M = 32
G = 64

NC = 2
NS = 16
NW = NC * NS

B_EDGE = 80
ECH = E // B_EDGE
WCH = 128
N_PAD = 10240
RPT = N_PAD // NS
ZROWS = B_EDGE
GCH = 16
NGRP = WCH // GCH
NBUF = 4
NITER = WCH // NBUF

TBLK = 5000
NBLK = N // TBLK


def _sc_scatter_round(msg, src2d, dst2d):
    mesh = plsc.VectorSubcoreMesh(core_axis_name="c", subcore_axis_name="s")

    @functools.partial(
        pl.kernel,
        out_type=jax.ShapeDtypeStruct((NC, N_PAD, S), jnp.float32),
        mesh=mesh,
        scratch_types=(
            [pltpu.VMEM((2, GCH, B_EDGE), jnp.int32),
             pltpu.VMEM((2, GCH, B_EDGE), jnp.int32)]
            + [pltpu.VMEM((B_EDGE, S), jnp.float32) for _ in range(NBUF)]
            + [pltpu.VMEM_SHARED((N_PAD, S), jnp.float32)]
            + [pltpu.SemaphoreType.DMA for _ in range(2 * NBUF + 1)]
        ),
    )
    def k(msg_hbm, src_hbm, dst_hbm, out_hbm, srcg, dstg, *rest):
        rows = rest[:NBUF]
        agg_sh = rest[NBUF]
        gsems = rest[NBUF + 1:2 * NBUF + 1]
        ssems = rest[2 * NBUF + 1:3 * NBUF + 1]
        isem = rest[3 * NBUF + 1]
        c = lax.axis_index("c")
        s = lax.axis_index("s")
        wid = c * NS + s
        base = wid * WCH
        wcount = jnp.minimum(WCH, ECH - base)
        niter = wcount // NBUF
        ngrp_w = wcount // GCH

        def zrow(i, _):
            def zcol(j, _):
                rows[0][i, pl.ds(j * 16, 16)] = jnp.zeros((16,), jnp.float32)
                return 0
            return lax.fori_loop(0, S // 16, zcol, 0)
        lax.fori_loop(0, B_EDGE, zrow, 0)

        pltpu.sync_copy(src_hbm.at[pl.ds(base, GCH)], srcg.at[0])
        pltpu.sync_copy(dst_hbm.at[pl.ds(base, GCH)], dstg.at[0])
        for b in range(1, NBUF):
            pltpu.async_copy(msg_hbm.at[srcg.at[0, b]], rows[b], gsems[b])

        for kk in range(RPT // ZROWS):
            pltpu.async_copy(rows[0],
                             agg_sh.at[pl.ds(s * RPT + kk * ZROWS, ZROWS)],
                             ssems[0])
        for kk in range(RPT // ZROWS):
            pltpu.make_async_copy(
                rows[0], agg_sh.at[pl.ds(s * RPT + kk * ZROWS, ZROWS)],
                ssems[0]).wait()
        pltpu.async_copy(msg_hbm.at[srcg.at[0, 0]], rows[0], gsems[0])
        plsc.subcore_barrier()

        def step(t, _):
            j0 = NBUF * t
            g = j0 // GCH
            slot = g % 2
            for b in range(NBUF):
                kk = (j0 + b) % GCH
                pltpu.make_async_copy(msg_hbm.at[srcg.at[slot, kk]], rows[b],
                                      gsems[b]).wait()
                pltpu.async_copy(rows[b], agg_sh.at[dstg.at[slot, kk]],
                                 ssems[b], add=True)

            @pl.when(j0 + NBUF < wcount)
            def _():
                jn0 = j0 + NBUF
                gn = jn0 // GCH
                slotn = gn % 2

                @pl.when(gn > g)
                def _():
                    nb = base + gn * GCH
                    pltpu.make_async_copy(src_hbm.at[pl.ds(nb, GCH)],
                                          srcg.at[slotn], isem).wait()
                    pltpu.make_async_copy(dst_hbm.at[pl.ds(nb, GCH)],
                                          dstg.at[slotn], isem).wait()

                for b in range(NBUF):
                    kk = (j0 + b) % GCH
                    kn = (jn0 + b) % GCH
                    pltpu.make_async_copy(rows[b],
                                          agg_sh.at[dstg.at[slot, kk]],
                                          ssems[b]).wait()
                    pltpu.async_copy(msg_hbm.at[srcg.at[slotn, kn]], rows[b],
                                     gsems[b])

            @pl.when(jnp.logical_and(j0 % GCH == 0, g + 1 < ngrp_w))
            def _():
                nb = base + (g + 1) * GCH
                pltpu.async_copy(src_hbm.at[pl.ds(nb, GCH)],
                                 srcg.at[(g + 1) % 2], isem)
                pltpu.async_copy(dst_hbm.at[pl.ds(nb, GCH)],
                                 dstg.at[(g + 1) % 2], isem)
            return 0
        lax.fori_loop(0, niter, step, 0)

        lslot = ((wcount - NBUF) // GCH) % 2
        for b in range(NBUF):
            pltpu.make_async_copy(
                rows[b], agg_sh.at[dstg.at[lslot, (wcount - NBUF + b) % GCH]],
                ssems[b]).wait()
        plsc.subcore_barrier()

        nwb = RPT // ZROWS

        def sl(kk):
            return pl.ds(s * RPT + kk * ZROWS, ZROWS)

        pltpu.async_copy(agg_sh.at[sl(0)], rows[0], gsems[0])
        for kk in range(nwb):
            b = kk % 2
            pltpu.make_async_copy(agg_sh.at[sl(kk)], rows[b], gsems[b]).wait()
            if kk + 1 < nwb:
                b2 = (kk + 1) % 2
                if kk >= 1:
                    pltpu.make_async_copy(rows[b2],
                                          out_hbm.at[c].at[sl(kk - 1)],
                                          ssems[b2]).wait()
                pltpu.async_copy(agg_sh.at[sl(kk + 1)], rows[b2], gsems[b2])
            pltpu.async_copy(rows[b], out_hbm.at[c].at[sl(kk)], ssems[b])
        for kk in (nwb - 2, nwb - 1):
            pltpu.make_async_copy(rows[kk % 2], out_hbm.at[c].at[sl(kk)],
                                  ssems[kk % 2]).wait()

    return k(msg, src2d, dst2d)


def _tc_encode(x, W_in, b_in, W_msg0, b_msg0):
    def body(x_ref, wi_ref, bi_ref, wm_ref, bm_ref, st_ref, msg_ref):
        st = jnp.maximum(
            jnp.dot(x_ref[...], wi_ref[...], preferred_element_type=jnp.float32)
            + bi_ref[...], 0.0)
        st_ref[...] = st
        msg_ref[...] = jnp.maximum(
            jnp.dot(st, wm_ref[...], preferred_element_type=jnp.float32)
            + bm_ref[...], 0.0)

    return pl.pallas_call(
        body,
        grid=(NBLK,),
        in_specs=[
            pl.BlockSpec((TBLK, D), lambda i: (i, 0)),
            pl.BlockSpec((D, S), lambda i: (0, 0)),
            pl.BlockSpec((1, S), lambda i: (0, 0)),
            pl.BlockSpec((S, S), lambda i: (0, 0)),
            pl.BlockSpec((1, S), lambda i: (0, 0)),
        ],
        out_specs=[
            pl.BlockSpec((TBLK, S), lambda i: (i, 0)),
            pl.BlockSpec((TBLK, S), lambda i: (i, 0)),
        ],
        out_shape=[
            jax.ShapeDtypeStruct((N, S), jnp.float32),
            jax.ShapeDtypeStruct((N, S), jnp.float32),
        ],
    )(x, W_in, b_in, W_msg0, b_msg0)


def _tc_update(agg, state, W_upd_r, b_upd_r, W_msg_n, b_msg_n):
    def body(agg_ref, st_ref, wu_ref, bu_ref, wm_ref, bm_ref,
             stn_ref, msg_ref):
        a = agg_ref[0] + agg_ref[1]
        h = st_ref[...] + jnp.maximum(
            jnp.dot(a, wu_ref[...], preferred_element_type=jnp.float32)
            + bu_ref[...], 0.0)
        stn_ref[...] = h
        msg_ref[...] = jnp.maximum(
            jnp.dot(h, wm_ref[...], preferred_element_type=jnp.float32)
            + bm_ref[...], 0.0)

    return pl.pallas_call(
        body,
        grid=(NBLK,),
        in_specs=[
            pl.BlockSpec((NC, TBLK, S), lambda i: (0, i, 0)),
            pl.BlockSpec((TBLK, S), lambda i: (i, 0)),
            pl.BlockSpec((S, S), lambda i: (0, 0)),
            pl.BlockSpec((1, S), lambda i: (0, 0)),
            pl.BlockSpec((S, S), lambda i: (0, 0)),
            pl.BlockSpec((1, S), lambda i: (0, 0)),
        ],
        out_specs=[
            pl.BlockSpec((TBLK, S), lambda i: (i, 0)),
            pl.BlockSpec((TBLK, S), lambda i: (i, 0)),
        ],
        out_shape=[
            jax.ShapeDtypeStruct((N, S), jnp.float32),
            jax.ShapeDtypeStruct((N, S), jnp.float32),
        ],
    )(agg, state, W_upd_r, b_upd_r, W_msg_n, b_msg_n)


def _tc_final(agg, state, W_upd_r, b_upd_r, batch3d, W_mean, b_mean, W_lv, b_lv):
    def body(agg_ref, st_ref, wu_ref, bu_ref, bat_ref, wm_ref, bm_ref,
             wl_ref, bl_ref, mean_ref, std_ref, acc_ref):
        i = pl.program_id(0)
        a = agg_ref[0] + agg_ref[1]
        h = st_ref[...] + jnp.maximum(
            jnp.dot(a, wu_ref[...], preferred_element_type=jnp.float32)
            + bu_ref[...], 0.0)
        bat = bat_ref[0]
        gids = lax.broadcasted_iota(jnp.int32, (G, TBLK), 0)
        P = (gids == bat).astype(jnp.float32)
        contrib = jnp.dot(P, h, preferred_element_type=jnp.float32)

        @pl.when(i == 0)
        def _():
            acc_ref[...] = contrib

        @pl.when(i > 0)
        def _():
            acc_ref[...] = acc_ref[...] + contrib

        @pl.when(i == NBLK - 1)
        def _():
            g = acc_ref[...]
            mean_ref[...] = (
                jnp.dot(g, wm_ref[...], preferred_element_type=jnp.float32)
                + bm_ref[...])
            lv = jnp.clip(
                jnp.dot(g, wl_ref[...], preferred_element_type=jnp.float32)
                + bl_ref[...], -20.0, 2.0)
            std_ref[...] = jnp.exp(0.5 * lv)

    return pl.pallas_call(
        body,
        grid=(NBLK,),
        in_specs=[
            pl.BlockSpec((NC, TBLK, S), lambda i: (0, i, 0)),
            pl.BlockSpec((TBLK, S), lambda i: (i, 0)),
            pl.BlockSpec((S, S), lambda i: (0, 0)),
            pl.BlockSpec((1, S), lambda i: (0, 0)),
            pl.BlockSpec((1, 1, TBLK), lambda i: (i, 0, 0)),
            pl.BlockSpec((S, M), lambda i: (0, 0)),
            pl.BlockSpec((1, M), lambda i: (0, 0)),
            pl.BlockSpec((S, M), lambda i: (0, 0)),
            pl.BlockSpec((1, M), lambda i: (0, 0)),
        ],
        out_specs=[
            pl.BlockSpec((G, M), lambda i: (0, 0)),
            pl.BlockSpec((G, M), lambda i: (0, 0)),
        ],
        out_shape=[
            jax.ShapeDtypeStruct((G, M), jnp.float32),
            jax.ShapeDtypeStruct((G, M), jnp.float32),
        ],
        scratch_shapes=[pltpu.VMEM((G, S), jnp.float32)],
        compiler_params=pltpu.CompilerParams(
            dimension_semantics=("arbitrary",)),
    )(agg, state, W_upd_r, b_upd_r, batch3d, W_mean, b_mean, W_lv, b_lv)


def kernel(x, edge_index, batch, W_in, b_in, W_msg, b_msg, W_upd, b_upd,
           W_mean, b_mean, W_lv, b_lv):
    src2d = edge_index[0].reshape(ECH, B_EDGE)
    dst2d = edge_index[1].reshape(ECH, B_EDGE)
    batch3d = batch.reshape(NBLK, 1, TBLK)

    b_in2 = b_in.reshape(1, S)
    b_msg2 = b_msg.reshape(R, 1, S)
    b_upd2 = b_upd.reshape(R, 1, S)

    state, msg = _tc_encode(x, W_in, b_in2, W_msg[0], b_msg2[0])
    for r in range(R - 1):
        agg = _sc_scatter_round(msg, src2d, dst2d)
        state, msg = _tc_update(agg, state, W_upd[r], b_upd2[r],
                                W_msg[r + 1], b_msg2[r + 1])
    agg = _sc_scatter_round(msg, src2d, dst2d)
    mean, std = _tc_final(agg, state, W_upd[R - 1], b_upd2[R - 1], batch3d,
                          W_mean, b_mean.reshape(1, M), W_lv,
                          b_lv.reshape(1, M))
    return (mean, std)
